# baseline (speedup 1.0000x reference)
import functools

import jax
import jax.numpy as jnp
import numpy as np
from jax import lax
from jax.experimental import pallas as pl
from jax.experimental.pallas import tpu as pltpu
from jax.experimental.pallas import tpu_sc as plsc

_N = 50000
_E = 800000
_Z = 4
_MS = 32
_MV = 16
_NB = 8
_L = 2
_RMAX = 5.0
_NORM = 1.0 / np.sqrt(16.0)

_NC, _NS = 2, 16
_NW = _NC * _NS
_EROWS = _E // 128
_EPAD = 6400
_NPAD = 50048

_EBLK = 1280
_NBLK = 2000


def _gather_body(F, tbl, idx2, out, idxall, rows0, rows1, sem0, sem1):
    c = lax.axis_index("c")
    s = lax.axis_index("s")
    w = s * _NC + c
    start = w * 200
    nk = jnp.clip(_EROWS - start, 0, 200)
    pltpu.sync_copy(idx2.at[pl.ds(start, 200)], idxall)

    def fire(k, buf, sem):
        pltpu.async_copy(tbl.at[idxall.at[k]], buf, sem)

    fire(0, rows0, sem0)
    fire(1, rows1, sem1)

    def body(kk, carry):
        for b, (buf, sem) in enumerate(((rows0, sem0), (rows1, sem1))):
            k = 2 * kk + b

            @pl.when(k < nk)
            def _():
                pltpu.make_async_copy(tbl.at[idxall.at[k]], buf, sem).wait()
                pltpu.sync_copy(buf, out.at[pl.ds((start + k) * 128, 128)])

            @pl.when(k + 2 < nk)
            def _():
                fire(k + 2, buf, sem)
        return carry

    lax.fori_loop(0, 100, body, 0)


@functools.lru_cache(maxsize=None)
def _make_gather(F, tc_tiling=True):
    mesh = plsc.VectorSubcoreMesh(
        core_axis_name="c", subcore_axis_name="s",
        num_cores=_NC, num_subcores=_NS)
    return pl.kernel(
        functools.partial(_gather_body, F),
        out_type=jax.ShapeDtypeStruct((_E, F), jnp.float32),
        mesh=mesh,
        compiler_params=pltpu.CompilerParams(use_tc_tiling_on_sc=tc_tiling),
        scratch_types=[
            pltpu.VMEM((200, 128), jnp.int32),
            pltpu.VMEM((128, F), jnp.float32),
            pltpu.VMEM((128, F), jnp.float32),
            pltpu.SemaphoreType.DMA,
            pltpu.SemaphoreType.DMA,
        ],
    )


def _scatter_body(M, idx2, out, idxall, upd0, upd1, zbuf, stage, shared,
                  sem0, sem1):
    c = lax.axis_index("c")
    s = lax.axis_index("s")
    start = s * 400
    ns = jnp.clip(_EROWS - start, 0, 400)
    pltpu.sync_copy(idx2.at[pl.ds(start, 400)], idxall)

    def zb(i, carry):
        zbuf[i, :] = jnp.zeros((16,), jnp.float32)
        return carry

    lax.fori_loop(0, 136, zb, 0)

    for g in range(3):
        cg = c * 3 + g
        for z in range(23):
            pltpu.sync_copy(zbuf, shared.at[pl.ds(s * 3128 + z * 136, 136)])
        plsc.subcore_barrier()

        def fire(k, buf, sem):
            pltpu.async_copy(
                M.at[cg, pl.ds((start + k) * 128, 128)], buf, sem)

        fire(0, upd0, sem0)
        fire(1, upd1, sem1)

        def body(kk, carry):
            for b, (buf, sem) in enumerate(((upd0, sem0), (upd1, sem1))):
                k = 2 * kk + b

                @pl.when(k < ns)
                def _():
                    pltpu.make_async_copy(
                        M.at[cg, pl.ds((start + k) * 128, 128)],
                        buf, sem).wait()
                    pltpu.sync_copy(buf, shared.at[idxall.at[k]], add=True)

                @pl.when(k + 2 < ns)
                def _():
                    fire(k + 2, buf, sem)
            return carry

        lax.fori_loop(0, 200, body, 0)
        plsc.subcore_barrier()
        for z in range(23):
            pltpu.sync_copy(shared.at[pl.ds(s * 3128 + z * 136, 136)], stage)
            pltpu.sync_copy(stage, out.at[cg, pl.ds(s * 3128 + z * 136, 136)])


@functools.lru_cache(maxsize=None)
def _make_scatter():
    mesh = plsc.VectorSubcoreMesh(
        core_axis_name="c", subcore_axis_name="s",
        num_cores=_NC, num_subcores=_NS)
    return pl.kernel(
        _scatter_body,
        out_type=jax.ShapeDtypeStruct((6, _NPAD, 16), jnp.float32),
        mesh=mesh,
        compiler_params=pltpu.CompilerParams(use_tc_tiling_on_sc=False),
        scratch_types=[
            pltpu.VMEM((400, 128), jnp.int32),
            pltpu.VMEM((128, 16), jnp.float32),
            pltpu.VMEM((128, 16), jnp.float32),
            pltpu.VMEM((136, 16), jnp.float32),
            pltpu.VMEM((136, 16), jnp.float32),
            pltpu.VMEM_SHARED((_NPAD, 16), jnp.float32),
            pltpu.SemaphoreType.DMA,
            pltpu.SemaphoreType.DMA,
        ],
    )


def _gather8(tbl, idx2):
    return _make_gather(8, tc_tiling=False)(tbl, idx2)


def _gather128(tbl, idx2):
    return _make_gather(128)(tbl, idx2)


def _scatter96(M, idx2):
    return _make_scatter()(M, idx2)


def _geom_body(ps_ref, pd_ref, cs_ref, cell_ref, emb_ref, sh_ref):
    shift = lax.dot_general(cs_ref[...], cell_ref[...], (((0,), (0,)), ((), ())),
                            preferred_element_type=jnp.float32)
    ev = pd_ref[...][:, :4] - ps_ref[...][:, :4] + shift
    r2 = jnp.sum(ev * ev, axis=1, keepdims=True)
    r = jnp.sqrt(r2)
    rs = jnp.maximum(r, 1e-9)
    sh_ref[...] = np.float32(np.sqrt(3.0)) * ev / rs
    x = r * np.float32(1.0 / _RMAX)
    x3 = x * x * x
    x6 = x3 * x3
    x7 = x6 * x
    x8 = x7 * x
    fc = 1.0 - 28.0 * x6 + 48.0 * x7 - 21.0 * x8
    fc = jnp.where(x < 1.0, fc, 0.0)
    nvec = (lax.broadcasted_iota(jnp.int32, (1, _NB), 1) + 1
            ).astype(jnp.float32) * np.float32(np.pi / _RMAX)
    arg = rs * nvec
    emb_ref[...] = np.float32(np.sqrt(2.0 / _RMAX)) * jnp.sin(arg) / rs * fc


def _geom(ps, pd, cs4, cell4):
    grid = (_E // _EBLK,)
    return pl.pallas_call(
        _geom_body,
        grid=grid,
        in_specs=[
            pl.BlockSpec((_EBLK, 8), lambda i: (i, 0)),
            pl.BlockSpec((_EBLK, 8), lambda i: (i, 0)),
            pl.BlockSpec((3, _EBLK), lambda i: (0, i)),
            pl.BlockSpec((3, 4), lambda i: (0, 0)),
        ],
        out_specs=[
            pl.BlockSpec((_EBLK, _NB), lambda i: (i, 0)),
            pl.BlockSpec((_EBLK, 4), lambda i: (i, 0)),
        ],
        out_shape=[
            jax.ShapeDtypeStruct((_E, _NB), jnp.float32),
            jax.ShapeDtypeStruct((_E, 4), jnp.float32),
        ],
    )(ps, pd, cs4, cell4)


def _msg_body(emb_ref, sh_ref, g_ref, wr1_ref, wr2_ref, wsv_ref, out_ref):
    emb = emb_ref[...]
    sh = sh_ref[...]
    G = g_ref[...]
    h = emb @ wr1_ref[...]
    h = h * jax.nn.sigmoid(h)
    wp = h @ wr2_ref[...]
    s_e = G[:, :_MS]
    sp_e = s_e @ wsv_ref[...]
    v0 = G[:, 32:48]
    v1 = G[:, 48:64]
    v2 = G[:, 64:80]
    sh0 = sh[:, 0:1]
    sh1 = sh[:, 1:2]
    sh2 = sh[:, 2:3]
    dot = v0 * sh0 + v1 * sh1 + v2 * sh2
    w_ss = wp[:, :32]
    w_vs = wp[:, 32:48]
    w_sv = wp[:, 48:64]
    w_vv = wp[:, 64:80]
    a = w_sv * sp_e
    parts = (w_ss[:, :16] * s_e[:, :16],
             w_ss[:, 16:32] * s_e[:, 16:32],
             w_vs * dot,
             a * sh0 + w_vv * v0,
             a * sh1 + w_vv * v1,
             a * sh2 + w_vv * v2)
    for g, p in enumerate(parts):
        out_ref[g] = jnp.concatenate(
            [p[160 * j:160 * (j + 1), :] for j in range(8)], axis=1)


def _msg(emb, sh, G, Wr1l, Wr2l, Wsvl):
    grid = (_E // _EBLK,)
    return pl.pallas_call(
        _msg_body,
        grid=grid,
        in_specs=[
            pl.BlockSpec((_EBLK, _NB), lambda i: (i, 0)),
            pl.BlockSpec((_EBLK, 4), lambda i: (i, 0)),
            pl.BlockSpec((_EBLK, 128), lambda i: (i, 0)),
            pl.BlockSpec((_NB, 32), lambda i: (0, 0)),
            pl.BlockSpec((32, 80), lambda i: (0, 0)),
            pl.BlockSpec((32, _MV), lambda i: (0, 0)),
        ],
        out_specs=pl.BlockSpec((6, _EBLK // 8, 128), lambda i: (0, i, 0)),
        out_shape=jax.ShapeDtypeStruct((6, _E // 8, 128), jnp.float32),
    )(emb, sh, G, Wr1l, Wr2l, Wsvl)


def _node_body(t_ref, sv_ref, oh_ref, wo_ref, wself_ref, wattr_ref,
               wvmix_ref, wgate_ref, out_ref):
    T = t_ref[...]
    s = T[:, :_MS]
    SV = sv_ref[...] * np.float32(_NORM)
    wo = wo_ref[...]
    s_new = (SV[0] @ wo[:16] + SV[1] @ wo[16:32] + SV[2] @ wo[32:48]
             + s @ wself_ref[...] + oh_ref[...] @ wattr_ref[...])
    s_new = s_new * jax.nn.sigmoid(s_new)
    gate = jax.nn.sigmoid(s @ wgate_ref[...])
    wv = wvmix_ref[...]
    parts = [s + s_new]
    for cdim in range(3):
        vc = T[:, 32 + 16 * cdim:48 + 16 * cdim]
        parts.append(vc + (SV[3 + cdim] @ wv) * gate)
    parts.append(T[:, 80:128])
    out_ref[...] = jnp.concatenate(parts, axis=1)


def _node(T, SV, oh, Wol, Wselfl, Wattrl, Wvmixl, Wgatel):
    grid = (_N // _NBLK,)
    return pl.pallas_call(
        _node_body,
        grid=grid,
        in_specs=[
            pl.BlockSpec((_NBLK, 128), lambda i: (i, 0)),
            pl.BlockSpec((6, _NBLK, 16), lambda i: (0, i, 0)),
            pl.BlockSpec((_NBLK, _Z), lambda i: (i, 0)),
            pl.BlockSpec((48, _MS), lambda i: (0, 0)),
            pl.BlockSpec((_MS, _MS), lambda i: (0, 0)),
            pl.BlockSpec((_Z, _MS), lambda i: (0, 0)),
            pl.BlockSpec((_MV, _MV), lambda i: (0, 0)),
            pl.BlockSpec((_MS, _MV), lambda i: (0, 0)),
        ],
        out_specs=pl.BlockSpec((_NBLK, 128), lambda i: (i, 0)),
        out_shape=jax.ShapeDtypeStruct((_N, 128), jnp.float32),
    )(T, SV, oh, Wol, Wselfl, Wattrl, Wvmixl, Wgatel)


def _init_body(oh_ref, w_ref, pos_ref, out_ref):
    s0 = oh_ref[...] @ w_ref[...]
    n = s0.shape[0]
    out_ref[...] = jnp.concatenate(
        [s0, jnp.zeros((n, 48), jnp.float32), pos_ref[...],
         jnp.zeros((n, 40), jnp.float32)], axis=1)


def _init(oh, Wlin_in, pos8):
    grid = (_N // _NBLK,)
    return pl.pallas_call(
        _init_body,
        grid=grid,
        in_specs=[
            pl.BlockSpec((_NBLK, _Z), lambda i: (i, 0)),
            pl.BlockSpec((_Z, _MS), lambda i: (0, 0)),
            pl.BlockSpec((_NBLK, 8), lambda i: (i, 0)),
        ],
        out_specs=pl.BlockSpec((_NBLK, 128), lambda i: (i, 0)),
        out_shape=jax.ShapeDtypeStruct((_N, 128), jnp.float32),
    )(oh, Wlin_in, pos8)


def _energy_body(t_ref, w1_ref, w2_ref, out_ref):
    feat = t_ref[...][:, :_MS] @ w1_ref[...]
    out_ref[...] = (feat @ w2_ref[...]) * 2.0


def _energy(T, Wlin1, Wlin2):
    grid = (_N // _NBLK,)
    return pl.pallas_call(
        _energy_body,
        grid=grid,
        in_specs=[
            pl.BlockSpec((_NBLK, 128), lambda i: (i, 0)),
            pl.BlockSpec((_MS, 16), lambda i: (0, 0)),
            pl.BlockSpec((16, 1), lambda i: (0, 0)),
        ],
        out_specs=pl.BlockSpec((_NBLK, 1), lambda i: (i, 0)),
        out_shape=jax.ShapeDtypeStruct((_N, 1), jnp.float32),
    )(T, Wlin1, Wlin2)


def kernel(atom_type, atom_pos, edge_src, edge_dst, edge_cell_shift, cell,
           image_index, Wlin_in, Wsv, Wr1, Wr2, Wo, Wself, Wattr, Wvmix,
           Wgate, Wlin1, Wlin2):
    pos8 = jnp.pad(atom_pos, ((0, 0), (0, 5)))
    cs_t = edge_cell_shift.T
    cell0 = cell.reshape(-1, 3, 3)[0]
    cell4 = jnp.pad(cell0, ((0, 0), (0, 1)))
    oh = (atom_type[:, None] ==
          jnp.arange(_Z, dtype=atom_type.dtype)[None, :]).astype(jnp.float32)
    idx_s = jnp.pad(edge_src.astype(jnp.int32).reshape(_EROWS, 128),
                    ((0, _EPAD - _EROWS), (0, 0)))
    idx_d = jnp.pad(edge_dst.astype(jnp.int32).reshape(_EROWS, 128),
                    ((0, _EPAD - _EROWS), (0, 0)))
    idx_t = jnp.pad(
        edge_dst.astype(jnp.int32).reshape(_E // _EBLK, 8, _EBLK // 8)
        .swapaxes(1, 2).reshape(_EROWS, 128),
        ((0, _EPAD - _EROWS), (0, 0)))

    del image_index
    pd = _gather8(pos8, idx_d)
    ps = _gather8(pos8, idx_s)
    emb, sh = _geom(ps, pd, cs_t, cell4)
    T = _init(oh, Wlin_in, pos8)
    for l in range(_L):
        G = _gather128(T, idx_s)
        M = _msg(emb, sh, G, Wr1[l], Wr2[l], Wsv[l])
        SV = _scatter96(M.reshape(6, _E, 16), idx_t)
        T = _node(T, SV, oh, Wo[l], Wself[l], Wattr[l], Wvmix[l], Wgate[l])
    return _energy(T, Wlin1, Wlin2)


# speedup vs baseline: 1.1401x; 1.1401x over previous
import functools

import jax
import jax.numpy as jnp
import numpy as np
from jax import lax
from jax.experimental import pallas as pl
from jax.experimental.pallas import tpu as pltpu
from jax.experimental.pallas import tpu_sc as plsc

_N = 50000
_E = 800000
_Z = 4
_MS = 32
_MV = 16
_NB = 8
_L = 2
_RMAX = 5.0
_NORM = 1.0 / np.sqrt(16.0)

_NC, _NS = 2, 16
_NW = _NC * _NS
_EROWS = _E // 128
_EPAD = 6400
_NPAD = 50048

_EBLK = 1280
_NBLK = 2000


def _gather_body(F, tbl, idx2, out, idxall, rows0, rows1, sem0, sem1):
    c = lax.axis_index("c")
    s = lax.axis_index("s")
    w = s * _NC + c
    start = w * 200
    nk = jnp.clip(_EROWS - start, 0, 200)
    pltpu.sync_copy(idx2.at[pl.ds(start, 200)], idxall)

    def fire(k, buf, sem):
        pltpu.async_copy(tbl.at[idxall.at[k]], buf, sem)

    fire(0, rows0, sem0)
    fire(1, rows1, sem1)

    def body(kk, carry):
        for b, (buf, sem) in enumerate(((rows0, sem0), (rows1, sem1))):
            k = 2 * kk + b

            @pl.when(k < nk)
            def _():
                pltpu.make_async_copy(tbl.at[idxall.at[k]], buf, sem).wait()
                pltpu.sync_copy(buf, out.at[pl.ds((start + k) * 128, 128)])

            @pl.when(k + 2 < nk)
            def _():
                fire(k + 2, buf, sem)
        return carry

    lax.fori_loop(0, 100, body, 0)


@functools.lru_cache(maxsize=None)
def _make_gather(F, tc_tiling=True):
    mesh = plsc.VectorSubcoreMesh(
        core_axis_name="c", subcore_axis_name="s",
        num_cores=_NC, num_subcores=_NS)
    return pl.kernel(
        functools.partial(_gather_body, F),
        out_type=jax.ShapeDtypeStruct((_E, F), jnp.float32),
        mesh=mesh,
        compiler_params=pltpu.CompilerParams(use_tc_tiling_on_sc=tc_tiling),
        scratch_types=[
            pltpu.VMEM((200, 128), jnp.int32),
            pltpu.VMEM((128, F), jnp.float32),
            pltpu.VMEM((128, F), jnp.float32),
            pltpu.SemaphoreType.DMA,
            pltpu.SemaphoreType.DMA,
        ],
    )


def _scatter_body(M, idx2, out, idxall, upd0, upd1, zbuf, stage, shared,
                  sem0, sem1):
    c = lax.axis_index("c")
    s = lax.axis_index("s")
    start = s * 400
    ns = jnp.clip(_EROWS - start, 0, 400)
    pltpu.sync_copy(idx2.at[pl.ds(start, 400)], idxall)

    def zb(i, carry):
        zbuf[i, :] = jnp.zeros((16,), jnp.float32)
        return carry

    lax.fori_loop(0, 136, zb, 0)

    for g in range(3):
        cg = c * 3 + g
        for z in range(23):
            pltpu.sync_copy(zbuf, shared.at[pl.ds(s * 3128 + z * 136, 136)])
        plsc.subcore_barrier()

        def fire(k, buf, sem):
            pltpu.async_copy(
                M.at[cg, pl.ds((start + k) * 128, 128)], buf, sem)

        fire(0, upd0, sem0)
        fire(1, upd1, sem1)

        def body(kk, carry):
            for b, (buf, sem) in enumerate(((upd0, sem0), (upd1, sem1))):
                k = 2 * kk + b

                @pl.when(k < ns)
                def _():
                    pltpu.make_async_copy(
                        M.at[cg, pl.ds((start + k) * 128, 128)],
                        buf, sem).wait()
                    pltpu.sync_copy(buf, shared.at[idxall.at[k]], add=True)

                @pl.when(k + 2 < ns)
                def _():
                    fire(k + 2, buf, sem)
            return carry

        lax.fori_loop(0, 200, body, 0)
        plsc.subcore_barrier()
        for z in range(23):
            pltpu.sync_copy(shared.at[pl.ds(s * 3128 + z * 136, 136)], stage)
            pltpu.sync_copy(stage, out.at[cg, pl.ds(s * 3128 + z * 136, 136)])


@functools.lru_cache(maxsize=None)
def _make_scatter():
    mesh = plsc.VectorSubcoreMesh(
        core_axis_name="c", subcore_axis_name="s",
        num_cores=_NC, num_subcores=_NS)
    return pl.kernel(
        _scatter_body,
        out_type=jax.ShapeDtypeStruct((6, _NPAD, 16), jnp.float32),
        mesh=mesh,
        compiler_params=pltpu.CompilerParams(use_tc_tiling_on_sc=False),
        scratch_types=[
            pltpu.VMEM((400, 128), jnp.int32),
            pltpu.VMEM((128, 16), jnp.float32),
            pltpu.VMEM((128, 16), jnp.float32),
            pltpu.VMEM((136, 16), jnp.float32),
            pltpu.VMEM((136, 16), jnp.float32),
            pltpu.VMEM_SHARED((_NPAD, 16), jnp.float32),
            pltpu.SemaphoreType.DMA,
            pltpu.SemaphoreType.DMA,
        ],
    )


def _gather8(tbl, idx2):
    return _make_gather(8, tc_tiling=False)(tbl, idx2)


def _gather128(tbl, idx2):
    return _make_gather(128)(tbl, idx2)


def _scatter96(M, idx2):
    return _make_scatter()(M, idx2)


def _geom_body(ps_ref, pd_ref, cs_ref, cell_ref, emb_ref, sh_ref):
    pst = ps_ref[...][:, :4].T
    pdt = pd_ref[...][:, :4].T
    shift = lax.dot_general(cell_ref[...], cs_ref[...], (((0,), (0,)), ((), ())),
                            preferred_element_type=jnp.float32)
    ev = pdt - pst + shift
    r2 = (ev[0:1] * ev[0:1] + ev[1:2] * ev[1:2] + ev[2:3] * ev[2:3])
    r = jnp.sqrt(r2)
    rs = jnp.maximum(r, 1e-9)
    inv = 1.0 / rs
    sh_ref[...] = (np.float32(np.sqrt(3.0)) * ev * inv).T
    x = r * np.float32(1.0 / _RMAX)
    x3 = x * x * x
    x6 = x3 * x3
    x7 = x6 * x
    x8 = x7 * x
    fc = 1.0 - 28.0 * x6 + 48.0 * x7 - 21.0 * x8
    fc = jnp.where(x < 1.0, fc, 0.0)
    nvec = (lax.broadcasted_iota(jnp.int32, (_NB, 1), 0) + 1
            ).astype(jnp.float32) * np.float32(np.pi / _RMAX)
    arg = rs * nvec
    emb_ref[...] = (np.float32(np.sqrt(2.0 / _RMAX)) * jnp.sin(arg)
                    * (inv * fc)).T


def _geom(ps, pd, cs4, cell4):
    grid = (_E // _EBLK,)
    return pl.pallas_call(
        _geom_body,
        grid=grid,
        in_specs=[
            pl.BlockSpec((_EBLK, 8), lambda i: (i, 0)),
            pl.BlockSpec((_EBLK, 8), lambda i: (i, 0)),
            pl.BlockSpec((3, _EBLK), lambda i: (0, i)),
            pl.BlockSpec((3, 4), lambda i: (0, 0)),
        ],
        out_specs=[
            pl.BlockSpec((_EBLK, _NB), lambda i: (i, 0)),
            pl.BlockSpec((_EBLK, 4), lambda i: (i, 0)),
        ],
        out_shape=[
            jax.ShapeDtypeStruct((_E, _NB), jnp.float32),
            jax.ShapeDtypeStruct((_E, 4), jnp.float32),
        ],
    )(ps, pd, cs4, cell4)


def _msg_body(emb_ref, sh_ref, g_ref, wr1_ref, wr2_ref, wsv_ref, out_ref):
    emb = emb_ref[...]
    sh = sh_ref[...]
    G = g_ref[...]
    h = emb @ wr1_ref[...]
    h = h * jax.nn.sigmoid(h)
    wp = h @ wr2_ref[...]
    s_e = G[:, :_MS]
    sp_e = s_e @ wsv_ref[...]
    v0 = G[:, 32:48]
    v1 = G[:, 48:64]
    v2 = G[:, 64:80]
    sh0 = sh[:, 0:1]
    sh1 = sh[:, 1:2]
    sh2 = sh[:, 2:3]
    dot = v0 * sh0 + v1 * sh1 + v2 * sh2
    w_ss = wp[:, :32]
    w_vs = wp[:, 32:48]
    w_sv = wp[:, 48:64]
    w_vv = wp[:, 64:80]
    a = w_sv * sp_e
    parts = (w_ss[:, :16] * s_e[:, :16],
             w_ss[:, 16:32] * s_e[:, 16:32],
             w_vs * dot,
             a * sh0 + w_vv * v0,
             a * sh1 + w_vv * v1,
             a * sh2 + w_vv * v2)
    for g, p in enumerate(parts):
        out_ref[g] = jnp.concatenate(
            [p[160 * j:160 * (j + 1), :] for j in range(8)], axis=1)


def _msg(emb, sh, G, Wr1l, Wr2l, Wsvl):
    grid = (_E // _EBLK,)
    return pl.pallas_call(
        _msg_body,
        grid=grid,
        in_specs=[
            pl.BlockSpec((_EBLK, _NB), lambda i: (i, 0)),
            pl.BlockSpec((_EBLK, 4), lambda i: (i, 0)),
            pl.BlockSpec((_EBLK, 128), lambda i: (i, 0)),
            pl.BlockSpec((_NB, 32), lambda i: (0, 0)),
            pl.BlockSpec((32, 80), lambda i: (0, 0)),
            pl.BlockSpec((32, _MV), lambda i: (0, 0)),
        ],
        out_specs=pl.BlockSpec((6, _EBLK // 8, 128), lambda i: (0, i, 0)),
        out_shape=jax.ShapeDtypeStruct((6, _E // 8, 128), jnp.float32),
    )(emb, sh, G, Wr1l, Wr2l, Wsvl)


def _node_body(t_ref, sv_ref, oh_ref, wo_ref, wself_ref, wattr_ref,
               wvmix_ref, wgate_ref, out_ref):
    T = t_ref[...]
    s = T[:, :_MS]
    SV = sv_ref[...] * np.float32(_NORM)
    wo = wo_ref[...]
    s_new = (SV[0] @ wo[:16] + SV[1] @ wo[16:32] + SV[2] @ wo[32:48]
             + s @ wself_ref[...] + oh_ref[...] @ wattr_ref[...])
    s_new = s_new * jax.nn.sigmoid(s_new)
    gate = jax.nn.sigmoid(s @ wgate_ref[...])
    wv = wvmix_ref[...]
    parts = [s + s_new]
    for cdim in range(3):
        vc = T[:, 32 + 16 * cdim:48 + 16 * cdim]
        parts.append(vc + (SV[3 + cdim] @ wv) * gate)
    parts.append(T[:, 80:128])
    out_ref[...] = jnp.concatenate(parts, axis=1)


def _node(T, SV, oh, Wol, Wselfl, Wattrl, Wvmixl, Wgatel):
    grid = (_N // _NBLK,)
    return pl.pallas_call(
        _node_body,
        grid=grid,
        in_specs=[
            pl.BlockSpec((_NBLK, 128), lambda i: (i, 0)),
            pl.BlockSpec((6, _NBLK, 16), lambda i: (0, i, 0)),
            pl.BlockSpec((_NBLK, _Z), lambda i: (i, 0)),
            pl.BlockSpec((48, _MS), lambda i: (0, 0)),
            pl.BlockSpec((_MS, _MS), lambda i: (0, 0)),
            pl.BlockSpec((_Z, _MS), lambda i: (0, 0)),
            pl.BlockSpec((_MV, _MV), lambda i: (0, 0)),
            pl.BlockSpec((_MS, _MV), lambda i: (0, 0)),
        ],
        out_specs=pl.BlockSpec((_NBLK, 128), lambda i: (i, 0)),
        out_shape=jax.ShapeDtypeStruct((_N, 128), jnp.float32),
    )(T, SV, oh, Wol, Wselfl, Wattrl, Wvmixl, Wgatel)


def _init_body(oh_ref, w_ref, pos_ref, out_ref):
    s0 = oh_ref[...] @ w_ref[...]
    n = s0.shape[0]
    out_ref[...] = jnp.concatenate(
        [s0, jnp.zeros((n, 48), jnp.float32), pos_ref[...],
         jnp.zeros((n, 40), jnp.float32)], axis=1)


def _init(oh, Wlin_in, pos8):
    grid = (_N // _NBLK,)
    return pl.pallas_call(
        _init_body,
        grid=grid,
        in_specs=[
            pl.BlockSpec((_NBLK, _Z), lambda i: (i, 0)),
            pl.BlockSpec((_Z, _MS), lambda i: (0, 0)),
            pl.BlockSpec((_NBLK, 8), lambda i: (i, 0)),
        ],
        out_specs=pl.BlockSpec((_NBLK, 128), lambda i: (i, 0)),
        out_shape=jax.ShapeDtypeStruct((_N, 128), jnp.float32),
    )(oh, Wlin_in, pos8)


def _energy_body(t_ref, w1_ref, w2_ref, out_ref):
    feat = t_ref[...][:, :_MS] @ w1_ref[...]
    out_ref[...] = (feat @ w2_ref[...]) * 2.0


def _energy(T, Wlin1, Wlin2):
    grid = (_N // _NBLK,)
    return pl.pallas_call(
        _energy_body,
        grid=grid,
        in_specs=[
            pl.BlockSpec((_NBLK, 128), lambda i: (i, 0)),
            pl.BlockSpec((_MS, 16), lambda i: (0, 0)),
            pl.BlockSpec((16, 1), lambda i: (0, 0)),
        ],
        out_specs=pl.BlockSpec((_NBLK, 1), lambda i: (i, 0)),
        out_shape=jax.ShapeDtypeStruct((_N, 1), jnp.float32),
    )(T, Wlin1, Wlin2)


def kernel(atom_type, atom_pos, edge_src, edge_dst, edge_cell_shift, cell,
           image_index, Wlin_in, Wsv, Wr1, Wr2, Wo, Wself, Wattr, Wvmix,
           Wgate, Wlin1, Wlin2):
    pos8 = jnp.pad(atom_pos, ((0, 0), (0, 5)))
    cs_t = edge_cell_shift.T
    cell0 = cell.reshape(-1, 3, 3)[0]
    cell4 = jnp.pad(cell0, ((0, 0), (0, 1)))
    oh = (atom_type[:, None] ==
          jnp.arange(_Z, dtype=atom_type.dtype)[None, :]).astype(jnp.float32)
    idx_s = jnp.pad(edge_src.astype(jnp.int32).reshape(_EROWS, 128),
                    ((0, _EPAD - _EROWS), (0, 0)))
    idx_d = jnp.pad(edge_dst.astype(jnp.int32).reshape(_EROWS, 128),
                    ((0, _EPAD - _EROWS), (0, 0)))
    idx_t = jnp.pad(
        edge_dst.astype(jnp.int32).reshape(_E // _EBLK, 8, _EBLK // 8)
        .swapaxes(1, 2).reshape(_EROWS, 128),
        ((0, _EPAD - _EROWS), (0, 0)))

    del image_index
    pd = _gather8(pos8, idx_d)
    ps = _gather8(pos8, idx_s)
    emb, sh = _geom(ps, pd, cs_t, cell4)
    T = _init(oh, Wlin_in, pos8)
    for l in range(_L):
        G = _gather128(T, idx_s)
        M = _msg(emb, sh, G, Wr1[l], Wr2[l], Wsv[l])
        SV = _scatter96(M.reshape(6, _E, 16), idx_t)
        T = _node(T, SV, oh, Wo[l], Wself[l], Wattr[l], Wvmix[l], Wgate[l])
    return _energy(T, Wlin1, Wlin2)


# speedup vs baseline: 1.1986x; 1.0513x over previous
import functools

import jax
import jax.numpy as jnp
import numpy as np
from jax import lax
from jax.experimental import pallas as pl
from jax.experimental.pallas import tpu as pltpu
from jax.experimental.pallas import tpu_sc as plsc

_N = 50000
_E = 800000
_Z = 4
_MS = 32
_MV = 16
_NB = 8
_L = 2
_RMAX = 5.0
_NORM = 1.0 / np.sqrt(16.0)

_NC, _NS = 2, 16
_NW = _NC * _NS
_EROWS = _E // 128
_EPAD = 6400
_NPAD = 50048

_EBLK = 3200
_NBLK = 2000


def _gather_body(F, tbl, idx2, out, idxall, rows0, rows1, sem0, sem1):
    c = lax.axis_index("c")
    s = lax.axis_index("s")
    w = s * _NC + c
    start = w * 200
    nk = jnp.clip(_EROWS - start, 0, 200)
    pltpu.sync_copy(idx2.at[pl.ds(start, 200)], idxall)

    def fire(k, buf, sem):
        pltpu.async_copy(tbl.at[idxall.at[k]], buf, sem)

    fire(0, rows0, sem0)
    fire(1, rows1, sem1)

    def body(kk, carry):
        for b, (buf, sem) in enumerate(((rows0, sem0), (rows1, sem1))):
            k = 2 * kk + b

            @pl.when(k < nk)
            def _():
                pltpu.make_async_copy(tbl.at[idxall.at[k]], buf, sem).wait()
                pltpu.sync_copy(buf, out.at[pl.ds((start + k) * 128, 128)])

            @pl.when(k + 2 < nk)
            def _():
                fire(k + 2, buf, sem)
        return carry

    lax.fori_loop(0, 100, body, 0)


@functools.lru_cache(maxsize=None)
def _make_gather(F, tc_tiling=True):
    mesh = plsc.VectorSubcoreMesh(
        core_axis_name="c", subcore_axis_name="s",
        num_cores=_NC, num_subcores=_NS)
    return pl.kernel(
        functools.partial(_gather_body, F),
        out_type=jax.ShapeDtypeStruct((_E, F), jnp.float32),
        mesh=mesh,
        compiler_params=pltpu.CompilerParams(use_tc_tiling_on_sc=tc_tiling),
        scratch_types=[
            pltpu.VMEM((200, 128), jnp.int32),
            pltpu.VMEM((128, F), jnp.float32),
            pltpu.VMEM((128, F), jnp.float32),
            pltpu.SemaphoreType.DMA,
            pltpu.SemaphoreType.DMA,
        ],
    )


def _scatter_body(M, idx2, out, idxall, upd0, upd1, zbuf, stage, shared,
                  sem0, sem1):
    c = lax.axis_index("c")
    s = lax.axis_index("s")
    start = s * 400
    ns = jnp.clip(_EROWS - start, 0, 400)
    pltpu.sync_copy(idx2.at[pl.ds(start, 400)], idxall)

    def zb(i, carry):
        zbuf[i, :] = jnp.zeros((16,), jnp.float32)
        return carry

    lax.fori_loop(0, 136, zb, 0)

    for g in range(3):
        cg = c * 3 + g
        for z in range(23):
            pltpu.sync_copy(zbuf, shared.at[pl.ds(s * 3128 + z * 136, 136)])
        plsc.subcore_barrier()

        def fire(k, buf, sem):
            pltpu.async_copy(
                M.at[cg, pl.ds((start + k) * 128, 128)], buf, sem)

        fire(0, upd0, sem0)
        fire(1, upd1, sem1)

        def body(kk, carry):
            for b, (buf, sem) in enumerate(((upd0, sem0), (upd1, sem1))):
                k = 2 * kk + b

                @pl.when(k < ns)
                def _():
                    pltpu.make_async_copy(
                        M.at[cg, pl.ds((start + k) * 128, 128)],
                        buf, sem).wait()
                    pltpu.sync_copy(buf, shared.at[idxall.at[k]], add=True)

                @pl.when(k + 2 < ns)
                def _():
                    fire(k + 2, buf, sem)
            return carry

        lax.fori_loop(0, 200, body, 0)
        plsc.subcore_barrier()
        for z in range(23):
            pltpu.sync_copy(shared.at[pl.ds(s * 3128 + z * 136, 136)], stage)
            pltpu.sync_copy(stage, out.at[cg, pl.ds(s * 3128 + z * 136, 136)])


@functools.lru_cache(maxsize=None)
def _make_scatter():
    mesh = plsc.VectorSubcoreMesh(
        core_axis_name="c", subcore_axis_name="s",
        num_cores=_NC, num_subcores=_NS)
    return pl.kernel(
        _scatter_body,
        out_type=jax.ShapeDtypeStruct((6, _NPAD, 16), jnp.float32),
        mesh=mesh,
        compiler_params=pltpu.CompilerParams(use_tc_tiling_on_sc=False),
        scratch_types=[
            pltpu.VMEM((400, 128), jnp.int32),
            pltpu.VMEM((128, 16), jnp.float32),
            pltpu.VMEM((128, 16), jnp.float32),
            pltpu.VMEM((136, 16), jnp.float32),
            pltpu.VMEM((136, 16), jnp.float32),
            pltpu.VMEM_SHARED((_NPAD, 16), jnp.float32),
            pltpu.SemaphoreType.DMA,
            pltpu.SemaphoreType.DMA,
        ],
    )


def _gather8(tbl, idx2):
    return _make_gather(8, tc_tiling=False)(tbl, idx2)


def _gather128(tbl, idx2):
    return _make_gather(128)(tbl, idx2)


def _scatter96(M, idx2):
    return _make_scatter()(M, idx2)


def _geom_body(ps_ref, pd_ref, cs_ref, cell_ref, emb_ref, sh_ref):
    pst = ps_ref[...][:, :4].T
    pdt = pd_ref[...][:, :4].T
    shift = lax.dot_general(cell_ref[...], cs_ref[...], (((0,), (0,)), ((), ())),
                            preferred_element_type=jnp.float32)
    ev = pdt - pst + shift
    r2 = (ev[0:1] * ev[0:1] + ev[1:2] * ev[1:2] + ev[2:3] * ev[2:3])
    r = jnp.sqrt(r2)
    rs = jnp.maximum(r, 1e-9)
    inv = 1.0 / rs
    sh_ref[...] = (np.float32(np.sqrt(3.0)) * ev * inv).T
    x = r * np.float32(1.0 / _RMAX)
    x3 = x * x * x
    x6 = x3 * x3
    x7 = x6 * x
    x8 = x7 * x
    fc = 1.0 - 28.0 * x6 + 48.0 * x7 - 21.0 * x8
    fc = jnp.where(x < 1.0, fc, 0.0)
    nvec = (lax.broadcasted_iota(jnp.int32, (_NB, 1), 0) + 1
            ).astype(jnp.float32) * np.float32(np.pi / _RMAX)
    arg = rs * nvec
    emb_ref[...] = (np.float32(np.sqrt(2.0 / _RMAX)) * jnp.sin(arg)
                    * (inv * fc)).T


def _geom(ps, pd, cs4, cell4):
    grid = (_E // _EBLK,)
    return pl.pallas_call(
        _geom_body,
        grid=grid,
        in_specs=[
            pl.BlockSpec((_EBLK, 8), lambda i: (i, 0)),
            pl.BlockSpec((_EBLK, 8), lambda i: (i, 0)),
            pl.BlockSpec((3, _EBLK), lambda i: (0, i)),
            pl.BlockSpec((3, 4), lambda i: (0, 0)),
        ],
        out_specs=[
            pl.BlockSpec((_EBLK, _NB), lambda i: (i, 0)),
            pl.BlockSpec((_EBLK, 4), lambda i: (i, 0)),
        ],
        out_shape=[
            jax.ShapeDtypeStruct((_E, _NB), jnp.float32),
            jax.ShapeDtypeStruct((_E, 4), jnp.float32),
        ],
    )(ps, pd, cs4, cell4)


def _msg_body(emb_ref, sh_ref, g_ref, wr1_ref, wr2_ref, wsv_ref, out_ref):
    emb = emb_ref[...]
    sh = sh_ref[...]
    G = g_ref[...]
    h = emb @ wr1_ref[...]
    h = h * jax.nn.sigmoid(h)
    wp = h @ wr2_ref[...]
    s_e = G[:, :_MS]
    sp_e = s_e @ wsv_ref[...]
    v0 = G[:, 32:48]
    v1 = G[:, 48:64]
    v2 = G[:, 64:80]
    sh0 = sh[:, 0:1]
    sh1 = sh[:, 1:2]
    sh2 = sh[:, 2:3]
    dot = v0 * sh0 + v1 * sh1 + v2 * sh2
    w_ss = wp[:, :32]
    w_vs = wp[:, 32:48]
    w_sv = wp[:, 48:64]
    w_vv = wp[:, 64:80]
    a = w_sv * sp_e
    parts = (w_ss[:, :16] * s_e[:, :16],
             w_ss[:, 16:32] * s_e[:, 16:32],
             w_vs * dot,
             a * sh0 + w_vv * v0,
             a * sh1 + w_vv * v1,
             a * sh2 + w_vv * v2)
    for g, p in enumerate(parts):
        q = _EBLK // 8
        out_ref[g] = jnp.concatenate(
            [p[q * j:q * (j + 1), :] for j in range(8)], axis=1)


def _msg(emb, sh, G, Wr1l, Wr2l, Wsvl):
    grid = (_E // _EBLK,)
    return pl.pallas_call(
        _msg_body,
        grid=grid,
        in_specs=[
            pl.BlockSpec((_EBLK, _NB), lambda i: (i, 0)),
            pl.BlockSpec((_EBLK, 4), lambda i: (i, 0)),
            pl.BlockSpec((_EBLK, 128), lambda i: (i, 0)),
            pl.BlockSpec((_NB, 32), lambda i: (0, 0)),
            pl.BlockSpec((32, 80), lambda i: (0, 0)),
            pl.BlockSpec((32, _MV), lambda i: (0, 0)),
        ],
        out_specs=pl.BlockSpec((6, _EBLK // 8, 128), lambda i: (0, i, 0)),
        out_shape=jax.ShapeDtypeStruct((6, _E // 8, 128), jnp.float32),
    )(emb, sh, G, Wr1l, Wr2l, Wsvl)


def _node_body(t_ref, sv_ref, oh_ref, wo_ref, wself_ref, wattr_ref,
               wvmix_ref, wgate_ref, out_ref):
    T = t_ref[...]
    s = T[:, :_MS]
    SV = sv_ref[...] * np.float32(_NORM)
    wo = wo_ref[...]
    s_new = (SV[0] @ wo[:16] + SV[1] @ wo[16:32] + SV[2] @ wo[32:48]
             + s @ wself_ref[...] + oh_ref[...] @ wattr_ref[...])
    s_new = s_new * jax.nn.sigmoid(s_new)
    gate = jax.nn.sigmoid(s @ wgate_ref[...])
    wv = wvmix_ref[...]
    parts = [s + s_new]
    for cdim in range(3):
        vc = T[:, 32 + 16 * cdim:48 + 16 * cdim]
        parts.append(vc + (SV[3 + cdim] @ wv) * gate)
    parts.append(T[:, 80:128])
    out_ref[...] = jnp.concatenate(parts, axis=1)


def _node(T, SV, oh, Wol, Wselfl, Wattrl, Wvmixl, Wgatel):
    grid = (_N // _NBLK,)
    return pl.pallas_call(
        _node_body,
        grid=grid,
        in_specs=[
            pl.BlockSpec((_NBLK, 128), lambda i: (i, 0)),
            pl.BlockSpec((6, _NBLK, 16), lambda i: (0, i, 0)),
            pl.BlockSpec((_NBLK, _Z), lambda i: (i, 0)),
            pl.BlockSpec((48, _MS), lambda i: (0, 0)),
            pl.BlockSpec((_MS, _MS), lambda i: (0, 0)),
            pl.BlockSpec((_Z, _MS), lambda i: (0, 0)),
            pl.BlockSpec((_MV, _MV), lambda i: (0, 0)),
            pl.BlockSpec((_MS, _MV), lambda i: (0, 0)),
        ],
        out_specs=pl.BlockSpec((_NBLK, 128), lambda i: (i, 0)),
        out_shape=jax.ShapeDtypeStruct((_N, 128), jnp.float32),
    )(T, SV, oh, Wol, Wselfl, Wattrl, Wvmixl, Wgatel)


def _init_body(oh_ref, w_ref, pos_ref, out_ref):
    s0 = oh_ref[...] @ w_ref[...]
    n = s0.shape[0]
    out_ref[...] = jnp.concatenate(
        [s0, jnp.zeros((n, 48), jnp.float32), pos_ref[...],
         jnp.zeros((n, 40), jnp.float32)], axis=1)


def _init(oh, Wlin_in, pos8):
    grid = (_N // _NBLK,)
    return pl.pallas_call(
        _init_body,
        grid=grid,
        in_specs=[
            pl.BlockSpec((_NBLK, _Z), lambda i: (i, 0)),
            pl.BlockSpec((_Z, _MS), lambda i: (0, 0)),
            pl.BlockSpec((_NBLK, 8), lambda i: (i, 0)),
        ],
        out_specs=pl.BlockSpec((_NBLK, 128), lambda i: (i, 0)),
        out_shape=jax.ShapeDtypeStruct((_N, 128), jnp.float32),
    )(oh, Wlin_in, pos8)


def _energy_body(t_ref, w1_ref, w2_ref, out_ref):
    feat = t_ref[...][:, :_MS] @ w1_ref[...]
    out_ref[...] = (feat @ w2_ref[...]) * 2.0


def _energy(T, Wlin1, Wlin2):
    grid = (_N // _NBLK,)
    return pl.pallas_call(
        _energy_body,
        grid=grid,
        in_specs=[
            pl.BlockSpec((_NBLK, 128), lambda i: (i, 0)),
            pl.BlockSpec((_MS, 16), lambda i: (0, 0)),
            pl.BlockSpec((16, 1), lambda i: (0, 0)),
        ],
        out_specs=pl.BlockSpec((_NBLK, 1), lambda i: (i, 0)),
        out_shape=jax.ShapeDtypeStruct((_N, 1), jnp.float32),
    )(T, Wlin1, Wlin2)


def kernel(atom_type, atom_pos, edge_src, edge_dst, edge_cell_shift, cell,
           image_index, Wlin_in, Wsv, Wr1, Wr2, Wo, Wself, Wattr, Wvmix,
           Wgate, Wlin1, Wlin2):
    pos8 = jnp.pad(atom_pos, ((0, 0), (0, 5)))
    cs_t = edge_cell_shift.T
    cell0 = cell.reshape(-1, 3, 3)[0]
    cell4 = jnp.pad(cell0, ((0, 0), (0, 1)))
    oh = (atom_type[:, None] ==
          jnp.arange(_Z, dtype=atom_type.dtype)[None, :]).astype(jnp.float32)
    idx_s = jnp.pad(edge_src.astype(jnp.int32).reshape(_EROWS, 128),
                    ((0, _EPAD - _EROWS), (0, 0)))
    idx_d = jnp.pad(edge_dst.astype(jnp.int32).reshape(_EROWS, 128),
                    ((0, _EPAD - _EROWS), (0, 0)))
    idx_t = jnp.pad(
        edge_dst.astype(jnp.int32).reshape(_E // _EBLK, 8, _EBLK // 8)
        .swapaxes(1, 2).reshape(_EROWS, 128),
        ((0, _EPAD - _EROWS), (0, 0)))

    del image_index
    pd = _gather8(pos8, idx_d)
    ps = _gather8(pos8, idx_s)
    emb, sh = _geom(ps, pd, cs_t, cell4)
    T = _init(oh, Wlin_in, pos8)
    for l in range(_L):
        G = _gather128(T, idx_s)
        M = _msg(emb, sh, G, Wr1[l], Wr2[l], Wsv[l])
        SV = _scatter96(M.reshape(6, _E, 16), idx_t)
        T = _node(T, SV, oh, Wo[l], Wself[l], Wattr[l], Wvmix[l], Wgate[l])
    return _energy(T, Wlin1, Wlin2)


# speedup vs baseline: 1.2117x; 1.0109x over previous
import functools

import jax
import jax.numpy as jnp
import numpy as np
from jax import lax
from jax.experimental import pallas as pl
from jax.experimental.pallas import tpu as pltpu
from jax.experimental.pallas import tpu_sc as plsc

_N = 50000
_E = 800000
_Z = 4
_MS = 32
_MV = 16
_NB = 8
_L = 2
_RMAX = 5.0
_NORM = 1.0 / np.sqrt(16.0)

_NC, _NS = 2, 16
_NW = _NC * _NS
_EROWS = _E // 128
_EPAD = 6400
_NPAD = 50048

_EBLK = 6400
_NBLK = 2000


def _gather_body(F, tbl, idx2, out, idxall, rows0, rows1, sem0, sem1):
    c = lax.axis_index("c")
    s = lax.axis_index("s")
    w = s * _NC + c
    start = w * 200
    nk = jnp.clip(_EROWS - start, 0, 200)
    pltpu.sync_copy(idx2.at[pl.ds(start, 200)], idxall)

    def fire(k, buf, sem):
        pltpu.async_copy(tbl.at[idxall.at[k]], buf, sem)

    fire(0, rows0, sem0)
    fire(1, rows1, sem1)

    def body(kk, carry):
        for b, (buf, sem) in enumerate(((rows0, sem0), (rows1, sem1))):
            k = 2 * kk + b

            @pl.when(k < nk)
            def _():
                pltpu.make_async_copy(tbl.at[idxall.at[k]], buf, sem).wait()
                pltpu.sync_copy(buf, out.at[pl.ds((start + k) * 128, 128)])

            @pl.when(k + 2 < nk)
            def _():
                fire(k + 2, buf, sem)
        return carry

    lax.fori_loop(0, 100, body, 0)


@functools.lru_cache(maxsize=None)
def _make_gather(F, tc_tiling=True):
    mesh = plsc.VectorSubcoreMesh(
        core_axis_name="c", subcore_axis_name="s",
        num_cores=_NC, num_subcores=_NS)
    return pl.kernel(
        functools.partial(_gather_body, F),
        out_type=jax.ShapeDtypeStruct((_E, F), jnp.float32),
        mesh=mesh,
        compiler_params=pltpu.CompilerParams(use_tc_tiling_on_sc=tc_tiling),
        scratch_types=[
            pltpu.VMEM((200, 128), jnp.int32),
            pltpu.VMEM((128, F), jnp.float32),
            pltpu.VMEM((128, F), jnp.float32),
            pltpu.SemaphoreType.DMA,
            pltpu.SemaphoreType.DMA,
        ],
    )


def _scatter_body(M, idx2, out, idxall, upd0, upd1, zbuf, stage, shared,
                  sem0, sem1):
    c = lax.axis_index("c")
    s = lax.axis_index("s")
    start = s * 400
    ns = jnp.clip(_EROWS - start, 0, 400)
    pltpu.sync_copy(idx2.at[pl.ds(start, 400)], idxall)

    def zb(i, carry):
        zbuf[i, :] = jnp.zeros((16,), jnp.float32)
        return carry

    lax.fori_loop(0, 136, zb, 0)

    for g in range(3):
        cg = c * 3 + g
        for z in range(23):
            pltpu.sync_copy(zbuf, shared.at[pl.ds(s * 3128 + z * 136, 136)])
        plsc.subcore_barrier()

        def fire(k, buf, sem):
            pltpu.async_copy(
                M.at[cg, pl.ds((start + k) * 128, 128)], buf, sem)

        fire(0, upd0, sem0)
        fire(1, upd1, sem1)

        def body(kk, carry):
            for b, (buf, sem) in enumerate(((upd0, sem0), (upd1, sem1))):
                k = 2 * kk + b

                @pl.when(k < ns)
                def _():
                    pltpu.make_async_copy(
                        M.at[cg, pl.ds((start + k) * 128, 128)],
                        buf, sem).wait()
                    pltpu.sync_copy(buf, shared.at[idxall.at[k]], add=True)

                @pl.when(k + 2 < ns)
                def _():
                    fire(k + 2, buf, sem)
            return carry

        lax.fori_loop(0, 200, body, 0)
        plsc.subcore_barrier()
        for z in range(23):
            pltpu.sync_copy(shared.at[pl.ds(s * 3128 + z * 136, 136)], stage)
            pltpu.sync_copy(stage, out.at[cg, pl.ds(s * 3128 + z * 136, 136)])


@functools.lru_cache(maxsize=None)
def _make_scatter():
    mesh = plsc.VectorSubcoreMesh(
        core_axis_name="c", subcore_axis_name="s",
        num_cores=_NC, num_subcores=_NS)
    return pl.kernel(
        _scatter_body,
        out_type=jax.ShapeDtypeStruct((6, _NPAD, 16), jnp.float32),
        mesh=mesh,
        compiler_params=pltpu.CompilerParams(use_tc_tiling_on_sc=False),
        scratch_types=[
            pltpu.VMEM((400, 128), jnp.int32),
            pltpu.VMEM((128, 16), jnp.float32),
            pltpu.VMEM((128, 16), jnp.float32),
            pltpu.VMEM((136, 16), jnp.float32),
            pltpu.VMEM((136, 16), jnp.float32),
            pltpu.VMEM_SHARED((_NPAD, 16), jnp.float32),
            pltpu.SemaphoreType.DMA,
            pltpu.SemaphoreType.DMA,
        ],
    )


def _gather8(tbl, idx2):
    return _make_gather(8, tc_tiling=False)(tbl, idx2)


def _gather128(tbl, idx2):
    return _make_gather(128)(tbl, idx2)


def _scatter96(M, idx2):
    return _make_scatter()(M, idx2)


def _geom_body(ps_ref, pd_ref, cs_ref, cell_ref, emb_ref, sh_ref):
    pst = ps_ref[...][:, :4].T
    pdt = pd_ref[...][:, :4].T
    shift = lax.dot_general(cell_ref[...], cs_ref[...], (((0,), (0,)), ((), ())),
                            preferred_element_type=jnp.float32)
    ev = pdt - pst + shift
    r2 = (ev[0:1] * ev[0:1] + ev[1:2] * ev[1:2] + ev[2:3] * ev[2:3])
    r = jnp.sqrt(r2)
    rs = jnp.maximum(r, 1e-9)
    inv = 1.0 / rs
    sh_ref[...] = (np.float32(np.sqrt(3.0)) * ev * inv).T
    x = r * np.float32(1.0 / _RMAX)
    x3 = x * x * x
    x6 = x3 * x3
    x7 = x6 * x
    x8 = x7 * x
    fc = 1.0 - 28.0 * x6 + 48.0 * x7 - 21.0 * x8
    fc = jnp.where(x < 1.0, fc, 0.0)
    nvec = (lax.broadcasted_iota(jnp.int32, (_NB, 1), 0) + 1
            ).astype(jnp.float32) * np.float32(np.pi / _RMAX)
    arg = rs * nvec
    emb_ref[...] = (np.float32(np.sqrt(2.0 / _RMAX)) * jnp.sin(arg)
                    * (inv * fc)).T


def _geom(ps, pd, cs4, cell4):
    grid = (_E // _EBLK,)
    return pl.pallas_call(
        _geom_body,
        grid=grid,
        in_specs=[
            pl.BlockSpec((_EBLK, 8), lambda i: (i, 0)),
            pl.BlockSpec((_EBLK, 8), lambda i: (i, 0)),
            pl.BlockSpec((3, _EBLK), lambda i: (0, i)),
            pl.BlockSpec((3, 4), lambda i: (0, 0)),
        ],
        out_specs=[
            pl.BlockSpec((_EBLK, _NB), lambda i: (i, 0)),
            pl.BlockSpec((_EBLK, 4), lambda i: (i, 0)),
        ],
        out_shape=[
            jax.ShapeDtypeStruct((_E, _NB), jnp.float32),
            jax.ShapeDtypeStruct((_E, 4), jnp.float32),
        ],
    )(ps, pd, cs4, cell4)


def _msg_body(emb_ref, sh_ref, g_ref, wr1_ref, wr2_ref, wsv_ref, out_ref):
    emb = emb_ref[...]
    sh = sh_ref[...]
    G = g_ref[...]
    h = emb @ wr1_ref[...]
    h = h * jax.nn.sigmoid(h)
    wp = h @ wr2_ref[...]
    s_e = G[:, :_MS]
    sp_e = s_e @ wsv_ref[...]
    v0 = G[:, 32:48]
    v1 = G[:, 48:64]
    v2 = G[:, 64:80]
    sh0 = sh[:, 0:1]
    sh1 = sh[:, 1:2]
    sh2 = sh[:, 2:3]
    dot = v0 * sh0 + v1 * sh1 + v2 * sh2
    w_ss = wp[:, :32]
    w_vs = wp[:, 32:48]
    w_sv = wp[:, 48:64]
    w_vv = wp[:, 64:80]
    a = w_sv * sp_e
    parts = (w_ss[:, :16] * s_e[:, :16],
             w_ss[:, 16:32] * s_e[:, 16:32],
             w_vs * dot,
             a * sh0 + w_vv * v0,
             a * sh1 + w_vv * v1,
             a * sh2 + w_vv * v2)
    for g, p in enumerate(parts):
        q = _EBLK // 8
        out_ref[g] = jnp.concatenate(
            [p[q * j:q * (j + 1), :] for j in range(8)], axis=1)


def _msg(emb, sh, G, Wr1l, Wr2l, Wsvl):
    grid = (_E // _EBLK,)
    return pl.pallas_call(
        _msg_body,
        grid=grid,
        in_specs=[
            pl.BlockSpec((_EBLK, _NB), lambda i: (i, 0)),
            pl.BlockSpec((_EBLK, 4), lambda i: (i, 0)),
            pl.BlockSpec((_EBLK, 128), lambda i: (i, 0)),
            pl.BlockSpec((_NB, 32), lambda i: (0, 0)),
            pl.BlockSpec((32, 80), lambda i: (0, 0)),
            pl.BlockSpec((32, _MV), lambda i: (0, 0)),
        ],
        out_specs=pl.BlockSpec((6, _EBLK // 8, 128), lambda i: (0, i, 0)),
        out_shape=jax.ShapeDtypeStruct((6, _E // 8, 128), jnp.float32),
    )(emb, sh, G, Wr1l, Wr2l, Wsvl)


def _node_body(t_ref, sv_ref, oh_ref, wo_ref, wself_ref, wattr_ref,
               wvmix_ref, wgate_ref, out_ref):
    T = t_ref[...]
    s = T[:, :_MS]
    SV = sv_ref[...] * np.float32(_NORM)
    wo = wo_ref[...]
    s_new = (SV[0] @ wo[:16] + SV[1] @ wo[16:32] + SV[2] @ wo[32:48]
             + s @ wself_ref[...] + oh_ref[...] @ wattr_ref[...])
    s_new = s_new * jax.nn.sigmoid(s_new)
    gate = jax.nn.sigmoid(s @ wgate_ref[...])
    wv = wvmix_ref[...]
    parts = [s + s_new]
    for cdim in range(3):
        vc = T[:, 32 + 16 * cdim:48 + 16 * cdim]
        parts.append(vc + (SV[3 + cdim] @ wv) * gate)
    parts.append(T[:, 80:128])
    out_ref[...] = jnp.concatenate(parts, axis=1)


def _node(T, SV, oh, Wol, Wselfl, Wattrl, Wvmixl, Wgatel):
    grid = (_N // _NBLK,)
    return pl.pallas_call(
        _node_body,
        grid=grid,
        in_specs=[
            pl.BlockSpec((_NBLK, 128), lambda i: (i, 0)),
            pl.BlockSpec((6, _NBLK, 16), lambda i: (0, i, 0)),
            pl.BlockSpec((_NBLK, _Z), lambda i: (i, 0)),
            pl.BlockSpec((48, _MS), lambda i: (0, 0)),
            pl.BlockSpec((_MS, _MS), lambda i: (0, 0)),
            pl.BlockSpec((_Z, _MS), lambda i: (0, 0)),
            pl.BlockSpec((_MV, _MV), lambda i: (0, 0)),
            pl.BlockSpec((_MS, _MV), lambda i: (0, 0)),
        ],
        out_specs=pl.BlockSpec((_NBLK, 128), lambda i: (i, 0)),
        out_shape=jax.ShapeDtypeStruct((_N, 128), jnp.float32),
    )(T, SV, oh, Wol, Wselfl, Wattrl, Wvmixl, Wgatel)


def _init_body(oh_ref, w_ref, pos_ref, out_ref):
    s0 = oh_ref[...] @ w_ref[...]
    n = s0.shape[0]
    out_ref[...] = jnp.concatenate(
        [s0, jnp.zeros((n, 48), jnp.float32), pos_ref[...],
         jnp.zeros((n, 40), jnp.float32)], axis=1)


def _init(oh, Wlin_in, pos8):
    grid = (_N // _NBLK,)
    return pl.pallas_call(
        _init_body,
        grid=grid,
        in_specs=[
            pl.BlockSpec((_NBLK, _Z), lambda i: (i, 0)),
            pl.BlockSpec((_Z, _MS), lambda i: (0, 0)),
            pl.BlockSpec((_NBLK, 8), lambda i: (i, 0)),
        ],
        out_specs=pl.BlockSpec((_NBLK, 128), lambda i: (i, 0)),
        out_shape=jax.ShapeDtypeStruct((_N, 128), jnp.float32),
    )(oh, Wlin_in, pos8)


def _energy_body(t_ref, w1_ref, w2_ref, out_ref):
    feat = t_ref[...][:, :_MS] @ w1_ref[...]
    out_ref[...] = (feat @ w2_ref[...]) * 2.0


def _energy(T, Wlin1, Wlin2):
    grid = (_N // _NBLK,)
    return pl.pallas_call(
        _energy_body,
        grid=grid,
        in_specs=[
            pl.BlockSpec((_NBLK, 128), lambda i: (i, 0)),
            pl.BlockSpec((_MS, 16), lambda i: (0, 0)),
            pl.BlockSpec((16, 1), lambda i: (0, 0)),
        ],
        out_specs=pl.BlockSpec((_NBLK, 1), lambda i: (i, 0)),
        out_shape=jax.ShapeDtypeStruct((_N, 1), jnp.float32),
    )(T, Wlin1, Wlin2)


def kernel(atom_type, atom_pos, edge_src, edge_dst, edge_cell_shift, cell,
           image_index, Wlin_in, Wsv, Wr1, Wr2, Wo, Wself, Wattr, Wvmix,
           Wgate, Wlin1, Wlin2):
    pos8 = jnp.pad(atom_pos, ((0, 0), (0, 5)))
    cs_t = edge_cell_shift.T
    cell0 = cell.reshape(-1, 3, 3)[0]
    cell4 = jnp.pad(cell0, ((0, 0), (0, 1)))
    oh = (atom_type[:, None] ==
          jnp.arange(_Z, dtype=atom_type.dtype)[None, :]).astype(jnp.float32)
    idx_s = jnp.pad(edge_src.astype(jnp.int32).reshape(_EROWS, 128),
                    ((0, _EPAD - _EROWS), (0, 0)))
    idx_d = jnp.pad(edge_dst.astype(jnp.int32).reshape(_EROWS, 128),
                    ((0, _EPAD - _EROWS), (0, 0)))
    idx_t = jnp.pad(
        edge_dst.astype(jnp.int32).reshape(_E // _EBLK, 8, _EBLK // 8)
        .swapaxes(1, 2).reshape(_EROWS, 128),
        ((0, _EPAD - _EROWS), (0, 0)))

    del image_index
    pd = _gather8(pos8, idx_d)
    ps = _gather8(pos8, idx_s)
    emb, sh = _geom(ps, pd, cs_t, cell4)
    T = _init(oh, Wlin_in, pos8)
    for l in range(_L):
        G = _gather128(T, idx_s)
        M = _msg(emb, sh, G, Wr1[l], Wr2[l], Wsv[l])
        SV = _scatter96(M.reshape(6, _E, 16), idx_t)
        T = _node(T, SV, oh, Wo[l], Wself[l], Wattr[l], Wvmix[l], Wgate[l])
    return _energy(T, Wlin1, Wlin2)
